# baseline (speedup 1.0000x reference)
import jax
import jax.numpy as jnp
from jax import lax
from jax.experimental import pallas as pl
from jax.experimental.pallas import tpu as pltpu, tpu_sc as plsc

R, C = 128, 32768
SC_R = 64
TC_R = R - SC_R
NC, NS = 2, 16
NW = NC * NS
ROWS_PER_W = SC_R // NW
LANES = 16
NVEC = C // LANES
NCHUNK = 2
CELEM = C // NCHUNK
CVEC = NVEC // NCHUNK
NCHAIN = 4
SPANC = CVEC // NCHAIN
NBUF = 3
NQ = ROWS_PER_W * NCHUNK
SC_PER_CORE = NS * ROWS_PER_W
TC_CBLK = 8192
INT_MAX = 2147483647


def _argmax_body(x_hbm, out_hbm, b0, b1, b2, res_buf, stage_v, outc,
                 shared, sem0, sem1, sem2):
    c = lax.axis_index("c")
    s = lax.axis_index("s")
    wid = c * NS + s
    row0 = wid * ROWS_PER_W
    lane = lax.iota(jnp.int32, LANES)
    bufs = (b0, b1, b2)
    sems = (sem0, sem1, sem2)
    copies = [None] * NBUF

    def start(q):
        k, h = divmod(q, NCHUNK)
        cp = pltpu.make_async_copy(
            x_hbm.at[row0 + k, pl.ds(h * CELEM, CELEM)],
            bufs[q % NBUF], sems[q % NBUF])
        cp.start()
        copies[q % NBUF] = cp

    for q in range(NBUF):
        start(q)

    res = jnp.zeros((LANES,), jnp.int32)
    rbest = None
    rbiter = None
    for q in range(NQ):
        copies[q % NBUF].wait()
        buf = bufs[q % NBUF]
        k, h = divmod(q, NCHUNK)

        def step(i, carry, _buf=buf, _h=h):
            bests, biters = carry
            nb, ni = [], []
            for j in range(NCHAIN):
                vi = i + j * SPANC
                v = _buf[pl.ds(vi * LANES, LANES)]
                m = v > bests[j]
                nb.append(jnp.where(m, v, bests[j]))
                gvi = vi + _h * CVEC
                ni.append(jnp.where(m, jnp.full((LANES,), gvi, jnp.int32),
                                    biters[j]))
            return tuple(nb), tuple(ni)

        bests0 = tuple(jnp.full((LANES,), -jnp.inf, jnp.float32)
                       for _ in range(NCHAIN))
        biters0 = tuple(jnp.zeros((LANES,), jnp.int32)
                        for _ in range(NCHAIN))
        bests, biters = lax.fori_loop(0, SPANC, step, (bests0, biters0),
                                      unroll=4)

        if h == 0:
            rbest, rbiter = bests[0], biters[0]
            rest = range(1, NCHAIN)
        else:
            rest = range(NCHAIN)
        for j in rest:
            m = bests[j] > rbest
            rbest = jnp.where(m, bests[j], rbest)
            rbiter = jnp.where(m, biters[j], rbiter)

        if h == NCHUNK - 1:
            gmaxv = rbest
            for step2 in (1, 2, 4, 8):
                gmaxv = jnp.maximum(
                    gmaxv, gmaxv.at[lane ^ step2].get(mode="promise_in_bounds"))
            idx = rbiter * LANES + lane
            cand = jnp.where(rbest == gmaxv, idx,
                             jnp.full((LANES,), INT_MAX, jnp.int32))
            for step2 in (1, 2, 4, 8):
                cand = jnp.minimum(
                    cand, cand.at[lane ^ step2].get(mode="promise_in_bounds"))
            res = jnp.where(lane == k, cand, res)

        if q + NBUF < NQ:
            start(q + NBUF)

    res_buf[...] = res
    pltpu.sync_copy(res_buf, shared.at[pl.ds(s * LANES, LANES)])
    plsc.subcore_barrier()

    @pl.when(s == 0)
    def _():
        pltpu.sync_copy(shared, stage_v)
        lm = jnp.bitwise_and(lane, ROWS_PER_W - 1)
        subs_per_chunk = LANES // ROWS_PER_W
        for t in range(SC_PER_CORE // LANES):
            g = []
            for i in range(subs_per_chunk):
                w = stage_v[pl.ds((subs_per_chunk * t + i) * LANES, LANES)]
                g.append(w.at[lm].get(mode="promise_in_bounds"))
            sel = g[-1]
            for i in range(subs_per_chunk - 2, -1, -1):
                sel = jnp.where(lane < (i + 1) * ROWS_PER_W, g[i], sel)
            outc[pl.ds(t * LANES, LANES)] = sel
        pltpu.sync_copy(outc, out_hbm.at[pl.ds(c * SC_PER_CORE,
                                               SC_PER_CORE)])


def _tc_body(x_ref, out_ref, m_ref):
    j = pl.program_id(1)
    x = x_ref[...]
    col = lax.broadcasted_iota(jnp.int32, x.shape, 1) + j * TC_CBLK
    pm = jnp.max(x, axis=1, keepdims=True)
    pc = jnp.min(jnp.where(x == pm, col, INT_MAX), axis=1, keepdims=True)
    pm_b = jnp.broadcast_to(pm, out_ref.shape)
    pc_b = jnp.broadcast_to(pc, out_ref.shape)

    @pl.when(j == 0)
    def _():
        m_ref[...] = jnp.full_like(m_ref, -jnp.inf)
        out_ref[...] = jnp.zeros_like(out_ref)

    m_old = m_ref[...]
    take = pm_b > m_old
    out_ref[...] = jnp.where(take, pc_b, out_ref[...])
    m_ref[...] = jnp.where(take, pm_b, m_old)


def kernel(x):
    mesh = plsc.VectorSubcoreMesh(core_axis_name="c", subcore_axis_name="s")
    sc_out = pl.kernel(
        _argmax_body,
        out_type=jax.ShapeDtypeStruct((SC_R,), jnp.int32),
        mesh=mesh,
        scratch_types=[
            pltpu.VMEM((CELEM,), jnp.float32),
            pltpu.VMEM((CELEM,), jnp.float32),
            pltpu.VMEM((CELEM,), jnp.float32),
            pltpu.VMEM((LANES,), jnp.int32),
            pltpu.VMEM((NS * LANES,), jnp.int32),
            pltpu.VMEM((SC_PER_CORE,), jnp.int32),
            pltpu.VMEM_SHARED((NS * LANES,), jnp.int32),
            pltpu.SemaphoreType.DMA,
            pltpu.SemaphoreType.DMA,
            pltpu.SemaphoreType.DMA,
        ],
    )(x)

    BLK = 8
    tc_out = pl.pallas_call(
        _tc_body,
        grid=(TC_R // BLK, C // TC_CBLK),
        in_specs=[pl.BlockSpec((BLK, TC_CBLK),
                               lambda i, j: (i + SC_R // BLK, j))],
        out_specs=pl.BlockSpec((BLK, 128), lambda i, j: (i, 0)),
        out_shape=jax.ShapeDtypeStruct((TC_R, 128), jnp.int32),
        scratch_shapes=[pltpu.VMEM((BLK, 128), jnp.float32)],
    )(x)

    return jnp.concatenate([sc_out, tc_out[:, 0]])


# speedup vs baseline: 1.4670x; 1.4670x over previous
import jax
import jax.numpy as jnp
from jax import lax
from jax.experimental import pallas as pl
from jax.experimental.pallas import tpu as pltpu, tpu_sc as plsc

R, C = 128, 32768
SC_R = 64
TC_R = R - SC_R
NC, NS = 2, 16
NW = NC * NS
ROWS_PER_W = SC_R // NW
LANES = 16
NVEC = C // LANES
NCHUNK = 2
CELEM = C // NCHUNK
CVEC = NVEC // NCHUNK
NCHAIN = 4
SPANC = CVEC // NCHAIN
NBUF = 3
NQ = ROWS_PER_W * NCHUNK
SC_PER_CORE = NS * ROWS_PER_W
TC_CBLK = 8192
INT_MAX = 2147483647


def _argmax_body(x_hbm, out_hbm, b0, b1, b2, res_buf, stage_v, outc,
                 shared, sem0, sem1, sem2):
    c = lax.axis_index("c")
    s = lax.axis_index("s")
    wid = c * NS + s
    row0 = wid * ROWS_PER_W
    lane = lax.iota(jnp.int32, LANES)
    bufs = (b0, b1, b2)
    sems = (sem0, sem1, sem2)
    copies = [None] * NBUF

    def start(q):
        k, h = divmod(q, NCHUNK)
        cp = pltpu.make_async_copy(
            x_hbm.at[row0 + k, pl.ds(h * CELEM, CELEM)],
            bufs[q % NBUF], sems[q % NBUF])
        cp.start()
        copies[q % NBUF] = cp

    for q in range(NBUF):
        start(q)

    res = jnp.zeros((LANES,), jnp.int32)
    rbest = None
    rbiter = None
    for q in range(NQ):
        copies[q % NBUF].wait()
        buf = bufs[q % NBUF]
        k, h = divmod(q, NCHUNK)

        def step(i, carry, _buf=buf, _h=h):
            bests, biters = carry
            nb, ni = [], []
            for j in range(NCHAIN):
                vi = i + j * SPANC
                v = _buf[pl.ds(vi * LANES, LANES)]
                m = v > bests[j]
                nb.append(jnp.where(m, v, bests[j]))
                gvi = vi + _h * CVEC
                ni.append(jnp.where(m, jnp.full((LANES,), gvi, jnp.int32),
                                    biters[j]))
            return tuple(nb), tuple(ni)

        bests0 = tuple(jnp.full((LANES,), -jnp.inf, jnp.float32)
                       for _ in range(NCHAIN))
        biters0 = tuple(jnp.zeros((LANES,), jnp.int32)
                        for _ in range(NCHAIN))
        bests, biters = lax.fori_loop(0, SPANC, step, (bests0, biters0),
                                      unroll=4)

        if h == 0:
            rbest, rbiter = bests[0], biters[0]
            rest = range(1, NCHAIN)
        else:
            rest = range(NCHAIN)
        for j in rest:
            m = bests[j] > rbest
            rbest = jnp.where(m, bests[j], rbest)
            rbiter = jnp.where(m, biters[j], rbiter)

        if h == NCHUNK - 1:
            gmaxv = rbest
            for step2 in (1, 2, 4, 8):
                gmaxv = jnp.maximum(
                    gmaxv, gmaxv.at[lane ^ step2].get(mode="promise_in_bounds"))
            idx = rbiter * LANES + lane
            cand = jnp.where(rbest == gmaxv, idx,
                             jnp.full((LANES,), INT_MAX, jnp.int32))
            for step2 in (1, 2, 4, 8):
                cand = jnp.minimum(
                    cand, cand.at[lane ^ step2].get(mode="promise_in_bounds"))
            res = jnp.where(lane == k, cand, res)

        if q + NBUF < NQ:
            start(q + NBUF)

    res_buf[...] = res
    pltpu.sync_copy(res_buf, shared.at[pl.ds(s * LANES, LANES)])
    plsc.subcore_barrier()

    @pl.when(s == 0)
    def _():
        pltpu.sync_copy(shared, stage_v)
        lm = jnp.bitwise_and(lane, ROWS_PER_W - 1)
        subs_per_chunk = LANES // ROWS_PER_W
        for t in range(SC_PER_CORE // LANES):
            g = []
            for i in range(subs_per_chunk):
                w = stage_v[pl.ds((subs_per_chunk * t + i) * LANES, LANES)]
                g.append(w.at[lm].get(mode="promise_in_bounds"))
            sel = g[-1]
            for i in range(subs_per_chunk - 2, -1, -1):
                sel = jnp.where(lane < (i + 1) * ROWS_PER_W, g[i], sel)
            outc[pl.ds(t * LANES, LANES)] = sel
        pltpu.sync_copy(outc, out_hbm.at[pl.ds(c * SC_PER_CORE,
                                               SC_PER_CORE)])


def _tc_body(x_ref, out_ref):
    x = x_ref[...]
    col = lax.broadcasted_iota(jnp.int32, x.shape, 1)
    m = jnp.max(x, axis=1, keepdims=True)
    cand = jnp.where(x == m, col, INT_MAX)
    am = jnp.min(cand, axis=1)
    out_ref[...] = jnp.broadcast_to(am[:, None], out_ref.shape)


def kernel(x):
    mesh = plsc.VectorSubcoreMesh(core_axis_name="c", subcore_axis_name="s")
    sc_out = pl.kernel(
        _argmax_body,
        out_type=jax.ShapeDtypeStruct((SC_R,), jnp.int32),
        mesh=mesh,
        scratch_types=[
            pltpu.VMEM((CELEM,), jnp.float32),
            pltpu.VMEM((CELEM,), jnp.float32),
            pltpu.VMEM((CELEM,), jnp.float32),
            pltpu.VMEM((LANES,), jnp.int32),
            pltpu.VMEM((NS * LANES,), jnp.int32),
            pltpu.VMEM((SC_PER_CORE,), jnp.int32),
            pltpu.VMEM_SHARED((NS * LANES,), jnp.int32),
            pltpu.SemaphoreType.DMA,
            pltpu.SemaphoreType.DMA,
            pltpu.SemaphoreType.DMA,
        ],
    )(x)

    BLK = 16
    tc_out = pl.pallas_call(
        _tc_body,
        grid=(TC_R // BLK,),
        in_specs=[pl.BlockSpec((BLK, C), lambda i: (i + SC_R // BLK, 0))],
        out_specs=pl.BlockSpec((BLK, 128), lambda i: (i, 0)),
        out_shape=jax.ShapeDtypeStruct((TC_R, 128), jnp.int32),
        )(x)

    return jnp.concatenate([sc_out, tc_out[:, 0]])
